# baseline (speedup 1.0000x reference)
import functools
import math

import jax
import jax.numpy as jnp
from jax import lax
from jax.experimental import pallas as pl
from jax.experimental.pallas import tpu as pltpu
from jax.experimental.pallas import tpu_sc as plsc

N = 10000
E = 320000
D = 128

NC = 2
NS = 16
LANES = 16
NW = NC * NS

C = 56
CPW = 180
E_PAD = NW * CPW * C
NPAD = N + 80
NZB = N // C
NZT = N - NZB * C
WBI = (NZB + NS) // NS
VPR = D // LANES


def _mm_body(h_ref, w_ref, hp_ref):
    hp_ref[...] = jnp.dot(h_ref[...], w_ref[...],
                          preferred_element_type=jnp.float32)


def _combine_body(a_ref, p0_ref, p1_ref, o_ref):
    o_ref[...] = a_ref[0, 0] * (p0_ref[...] + p1_ref[...])


def _sc_body(hp_hbm, src_hbm, dst_hbm, out_hbm,
             hs0, hs1, hs2, hd0, hd1, hd2,
             si0, si1, si2, di0, di1, di2, dc0, dc1, dc2,
             acc,
             sg0, sg1, sg2, ss0, ss1, ss2,
             fi0, fi1, fi2, fd0, fd1, fd2):
    cid = lax.axis_index("c")
    sid = lax.axis_index("s")
    wid = (1 - cid) * NS + sid
    hs = (hs0, hs1, hs2)
    hd = (hd0, hd1, hd2)
    sidx = (si0, si1, si2)
    didx = (di0, di1, di2)
    dsct = (dc0, dc1, dc2)
    sg = (sg0, sg1, sg2)
    ss = (ss0, ss1, ss2)
    fi = (fi0, fi1, fi2)
    fd = (fd0, fd1, fd2)

    cb = wid * CPW

    def _idx_base(cn):
        return pl.multiple_of((cb + cn) * C, C)

    def _zrow(e, _):
        for j in range(VPR):
            hs0[e, pl.ds(j * LANES, LANES)] = jnp.zeros((LANES,), jnp.float32)
        return 0
    lax.fori_loop(0, C, _zrow, 0)
    for r in range(WBI):
        b = r * NS + sid

        @pl.when(b < NZB)
        def _():
            pltpu.sync_copy(hs0, acc.at[pl.ds(pl.multiple_of(b * C, C), C)])

    @pl.when(sid == 0)
    def _():
        pltpu.sync_copy(hs0.at[pl.ds(0, NZT)], acc.at[pl.ds(NZB * C, NZT)])
    plsc.subcore_barrier()

    def _fetch_gidx(cn, b):
        pltpu.async_copy(src_hbm.at[pl.ds(_idx_base(cn), C)], sidx[b], fi[b])
        pltpu.async_copy(dst_hbm.at[pl.ds(_idx_base(cn), C)], didx[b], fi[b])

    def _wait_gidx(cn, b):
        pltpu.make_async_copy(src_hbm.at[pl.ds(_idx_base(cn), C)], sidx[b],
                              fi[b]).wait()
        pltpu.make_async_copy(dst_hbm.at[pl.ds(_idx_base(cn), C)], didx[b],
                              fi[b]).wait()

    def _issue_gather(b):
        pltpu.async_copy(hp_hbm.at[sidx[b]], hs[b], sg[b])
        pltpu.async_copy(hp_hbm.at[didx[b]], hd[b], sg[b])

    def _wait_gather(b):
        pltpu.make_async_copy(hp_hbm.at[sidx[b]], hs[b], sg[b]).wait()
        pltpu.make_async_copy(hp_hbm.at[didx[b]], hd[b], sg[b]).wait()

    def _fetch_sidx(cn, b):
        pltpu.async_copy(dst_hbm.at[pl.ds(_idx_base(cn), C)], dsct[b], fd[b])

    def _wait_sidx(cn, b):
        pltpu.make_async_copy(dst_hbm.at[pl.ds(_idx_base(cn), C)], dsct[b],
                              fd[b]).wait()

    def _compute_chunk(b):
        hsb, hdb = hs[b], hd[b]

        @plsc.parallel_loop(0, C, 1, unroll=4)
        def _(e):
            vs = [hsb[e, pl.ds(j * LANES, LANES)] for j in range(VPR)]
            vd = [hdb[e, pl.ds(j * LANES, LANES)] for j in range(VPR)]
            acc_v = vs[0] * vd[0]
            for j in range(1, VPR):
                acc_v = acc_v + vs[j] * vd[j]
            lanes = lax.iota(jnp.int32, LANES)
            for shift in (8, 4, 2, 1):
                perm = jnp.bitwise_xor(lanes, shift)
                acc_v = acc_v + acc_v.at[perm].get(mode="promise_in_bounds")
            for j in range(VPR):
                hsb[e, pl.ds(j * LANES, LANES)] = vs[j] * acc_v

    _fetch_gidx(0, 0)
    _fetch_gidx(1, 1)
    _fetch_gidx(2, 2)
    _fetch_sidx(0, 0)
    _fetch_sidx(1, 1)
    _wait_gidx(0, 0)
    _wait_gidx(1, 1)
    _issue_gather(0)
    _issue_gather(1)

    def _outer(i, _):
        for p in range(3):
            b = p
            b2 = (p + 2) % 3
            g = 3 * i + p
            _wait_gather(b)

            @pl.when(g + 3 < CPW)
            def _():
                _fetch_gidx(g + 3, b)
            _compute_chunk(b)
            _wait_sidx(g, b)
            pltpu.async_copy(hs[b], acc.at[dsct[b]], ss[b], add=True)

            @pl.when(g + 2 < CPW)
            def _():
                if p == 0:
                    @pl.when(i > 0)
                    def _():
                        pltpu.make_async_copy(hs[b2], acc.at[dsct[b2]],
                                              ss[b2]).wait()
                else:
                    pltpu.make_async_copy(hs[b2], acc.at[dsct[b2]],
                                          ss[b2]).wait()
                _wait_gidx(g + 2, b2)
                _issue_gather(b2)
                _fetch_sidx(g + 2, b2)
        return 0

    lax.fori_loop(0, CPW // 3, _outer, 0)
    for b in range(3):
        pltpu.make_async_copy(hs[b], acc.at[dsct[b]], ss[b]).wait()

    plsc.subcore_barrier()
    for r in range(WBI):
        b = r * NS + sid

        @pl.when(b < NZB)
        def _():
            base = pl.multiple_of(b * C, C)
            pltpu.sync_copy(acc.at[pl.ds(base, C)],
                            out_hbm.at[cid, pl.ds(base, C)])

    @pl.when(sid == 0)
    def _():
        pltpu.sync_copy(acc.at[pl.ds(NZB * C, NZT)],
                        out_hbm.at[cid, pl.ds(NZB * C, NZT)])


def kernel(h, edge_index, W, attention_w):
    a = jnp.sum(attention_w).reshape(1, 1).astype(jnp.float32)
    src1 = jnp.concatenate(
        [edge_index[0], jnp.full((E_PAD - E,), N, dtype=jnp.int32)])
    dst1 = jnp.concatenate(
        [edge_index[1],
         jnp.arange(E_PAD - E, dtype=jnp.int32) % jnp.int32(N)])
    h_pad = jnp.pad(h, ((0, NPAD - N), (0, 0)))

    mm_grid = 10
    rows = NPAD // mm_grid
    hp = pl.pallas_call(
        _mm_body,
        grid=(mm_grid,),
        in_specs=[
            pl.BlockSpec((rows, D), lambda i: (i, 0)),
            pl.BlockSpec((D, D), lambda i: (0, 0)),
        ],
        out_specs=pl.BlockSpec((rows, D), lambda i: (i, 0)),
        out_shape=jax.ShapeDtypeStruct((NPAD, D), jnp.float32),
    )(h_pad, W)

    mesh = plsc.VectorSubcoreMesh(core_axis_name="c", subcore_axis_name="s",
                                  num_cores=NC, num_subcores=NS)
    partials = pl.kernel(
        _sc_body,
        out_type=jax.ShapeDtypeStruct((NC, N, D), jnp.float32),
        mesh=mesh,
        scratch_types=(
            [pltpu.VMEM((C, D), jnp.float32)] * 6
            + [pltpu.VMEM((C,), jnp.int32)] * 9
            + [pltpu.VMEM_SHARED((N, D), jnp.float32)]
            + [pltpu.SemaphoreType.DMA] * 12
        ),
    )(hp, src1, dst1)

    grid = 10
    rows_o = N // grid
    out = pl.pallas_call(
        _combine_body,
        grid=(grid,),
        in_specs=[
            pl.BlockSpec((1, 1), lambda i: (0, 0), memory_space=pltpu.SMEM),
            pl.BlockSpec((rows_o, D), lambda i: (i, 0)),
            pl.BlockSpec((rows_o, D), lambda i: (i, 0)),
        ],
        out_specs=pl.BlockSpec((rows_o, D), lambda i: (i, 0)),
        out_shape=jax.ShapeDtypeStruct((N, D), jnp.float32),
    )(a, partials[0], partials[1])
    return out


# speedup vs baseline: 1.4915x; 1.4915x over previous
import functools
import math

import jax
import jax.numpy as jnp
from jax import lax
from jax.experimental import pallas as pl
from jax.experimental.pallas import tpu as pltpu
from jax.experimental.pallas import tpu_sc as plsc

N = 10000
E = 320000
D = 128

NC = 2
NS = 16
LANES = 16

C = 64
TCH = E // C
T0 = 2500
NZB = N // C
NZT = N - NZB * C
WBI = (NZB + NS) // NS
VPR = D // LANES


def _mm_body(h_ref, w_ref, hp_ref):
    hp_ref[...] = jnp.dot(h_ref[...], w_ref[...],
                          preferred_element_type=jnp.float32)


def _combine_body(a_ref, p0_ref, p1_ref, o_ref):
    o_ref[...] = a_ref[0, 0] * (p0_ref[...] + p1_ref[...])


def _sc_body(hp_hbm, src_hbm, dst_hbm, out_hbm,
             hs0, hs1, hs2, hd0, hd1, hd2,
             si0, si1, si2, di0, di1, di2, dc0, dc1, dc2,
             acc,
             sg0, sg1, sg2, ss0, ss1, ss2,
             fi0, fi1, fi2, fd0, fd1, fd2):
    cid = lax.axis_index("c")
    sid = lax.axis_index("s")
    hs = (hs0, hs1, hs2)
    hd = (hd0, hd1, hd2)
    sidx = (si0, si1, si2)
    didx = (di0, di1, di2)
    dsct = (dc0, dc1, dc2)
    sg = (sg0, sg1, sg2)
    ss = (ss0, ss1, ss2)
    fi = (fi0, fi1, fi2)
    fd = (fd0, fd1, fd2)

    q0, r0 = divmod(T0, NS)
    q1, r1 = divmod(TCH - T0, NS)
    is0 = cid == 0
    my_n = jnp.where(is0,
                     q0 + (sid < r0).astype(jnp.int32),
                     q1 + (sid < r1).astype(jnp.int32))
    my_s = jnp.where(is0,
                     sid * q0 + jnp.minimum(sid, r0),
                     T0 + sid * q1 + jnp.minimum(sid, r1))

    def _idx_base(cn):
        return pl.multiple_of((my_s + cn) * C, C)

    def _zrow(e, _):
        for j in range(VPR):
            hs0[e, pl.ds(j * LANES, LANES)] = jnp.zeros((LANES,), jnp.float32)
        return 0
    lax.fori_loop(0, C, _zrow, 0)
    for r in range(WBI):
        b = r * NS + sid

        @pl.when(b < NZB)
        def _():
            pltpu.sync_copy(hs0, acc.at[pl.ds(pl.multiple_of(b * C, C), C)])

    @pl.when(sid == 0)
    def _():
        pltpu.sync_copy(hs0.at[pl.ds(0, NZT)], acc.at[pl.ds(NZB * C, NZT)])
    plsc.subcore_barrier()

    def _fetch_gidx(cn, b):
        pltpu.async_copy(src_hbm.at[pl.ds(_idx_base(cn), C)], sidx[b], fi[b])
        pltpu.async_copy(dst_hbm.at[pl.ds(_idx_base(cn), C)], didx[b], fi[b])

    def _wait_gidx(cn, b):
        pltpu.make_async_copy(src_hbm.at[pl.ds(_idx_base(cn), C)], sidx[b],
                              fi[b]).wait()
        pltpu.make_async_copy(dst_hbm.at[pl.ds(_idx_base(cn), C)], didx[b],
                              fi[b]).wait()

    def _issue_gather(b):
        pltpu.async_copy(hp_hbm.at[sidx[b]], hs[b], sg[b])
        pltpu.async_copy(hp_hbm.at[didx[b]], hd[b], sg[b])

    def _wait_gather(b):
        pltpu.make_async_copy(hp_hbm.at[sidx[b]], hs[b], sg[b]).wait()
        pltpu.make_async_copy(hp_hbm.at[didx[b]], hd[b], sg[b]).wait()

    def _fetch_sidx(cn, b):
        pltpu.async_copy(dst_hbm.at[pl.ds(_idx_base(cn), C)], dsct[b], fd[b])

    def _wait_sidx(cn, b):
        pltpu.make_async_copy(dst_hbm.at[pl.ds(_idx_base(cn), C)], dsct[b],
                              fd[b]).wait()

    def _compute_chunk(b):
        hsb, hdb = hs[b], hd[b]

        @plsc.parallel_loop(0, C, 1, unroll=4)
        def _(e):
            vs = [hsb[e, pl.ds(j * LANES, LANES)] for j in range(VPR)]
            vd = [hdb[e, pl.ds(j * LANES, LANES)] for j in range(VPR)]
            acc_v = vs[0] * vd[0]
            for j in range(1, VPR):
                acc_v = acc_v + vs[j] * vd[j]
            lanes = lax.iota(jnp.int32, LANES)
            for shift in (8, 4, 2, 1):
                perm = jnp.bitwise_xor(lanes, shift)
                acc_v = acc_v + acc_v.at[perm].get(mode="promise_in_bounds")
            for j in range(VPR):
                hsb[e, pl.ds(j * LANES, LANES)] = vs[j] * acc_v

    _fetch_gidx(0, 0)
    _fetch_gidx(1, 1)
    _fetch_gidx(2, 2)
    _fetch_sidx(0, 0)
    _fetch_sidx(1, 1)
    _wait_gidx(0, 0)
    _wait_gidx(1, 1)
    _issue_gather(0)
    _issue_gather(1)

    def _outer(i, _):
        for p in range(3):
            b = p
            b2 = (p + 2) % 3
            g = 3 * i + p

            @pl.when(g < my_n)
            def _():
                _wait_gather(b)

                @pl.when(g + 3 < my_n)
                def _():
                    _fetch_gidx(g + 3, b)
                _compute_chunk(b)
                _wait_sidx(g, b)
                pltpu.async_copy(hs[b], acc.at[dsct[b]], ss[b], add=True)

            @pl.when(g + 2 < my_n)
            def _():
                if p == 0:
                    @pl.when(i > 0)
                    def _():
                        pltpu.make_async_copy(hs[b2], acc.at[dsct[b2]],
                                              ss[b2]).wait()
                else:
                    pltpu.make_async_copy(hs[b2], acc.at[dsct[b2]],
                                          ss[b2]).wait()
                _wait_gidx(g + 2, b2)
                _issue_gather(b2)
                _fetch_sidx(g + 2, b2)
        return 0

    lax.fori_loop(0, (my_n + 2) // 3, _outer, 0)
    for b in range(3):
        pltpu.make_async_copy(hs[b], acc.at[dsct[b]], ss[b]).wait()

    plsc.subcore_barrier()
    for r in range(WBI):
        b = r * NS + sid

        @pl.when(b < NZB)
        def _():
            base = pl.multiple_of(b * C, C)
            pltpu.sync_copy(acc.at[pl.ds(base, C)],
                            out_hbm.at[cid, pl.ds(base, C)])

    @pl.when(sid == 0)
    def _():
        pltpu.sync_copy(acc.at[pl.ds(NZB * C, NZT)],
                        out_hbm.at[cid, pl.ds(NZB * C, NZT)])


def kernel(h, edge_index, W, attention_w):
    a = jnp.sum(attention_w).reshape(1, 1).astype(jnp.float32)
    src = edge_index[0]
    dst = edge_index[1]

    mm_grid = 10
    rows = N // mm_grid
    hp = pl.pallas_call(
        _mm_body,
        grid=(mm_grid,),
        in_specs=[
            pl.BlockSpec((rows, D), lambda i: (i, 0)),
            pl.BlockSpec((D, D), lambda i: (0, 0)),
        ],
        out_specs=pl.BlockSpec((rows, D), lambda i: (i, 0)),
        out_shape=jax.ShapeDtypeStruct((N, D), jnp.float32),
    )(h, W)

    mesh = plsc.VectorSubcoreMesh(core_axis_name="c", subcore_axis_name="s",
                                  num_cores=NC, num_subcores=NS)
    partials = pl.kernel(
        _sc_body,
        out_type=jax.ShapeDtypeStruct((NC, N, D), jnp.float32),
        mesh=mesh,
        scratch_types=(
            [pltpu.VMEM((C, D), jnp.float32)] * 6
            + [pltpu.VMEM((C,), jnp.int32)] * 9
            + [pltpu.VMEM_SHARED((N, D), jnp.float32)]
            + [pltpu.SemaphoreType.DMA] * 12
        ),
    )(hp, src, dst)

    grid = 10
    rows_o = N // grid
    out = pl.pallas_call(
        _combine_body,
        grid=(grid,),
        in_specs=[
            pl.BlockSpec((1, 1), lambda i: (0, 0), memory_space=pltpu.SMEM),
            pl.BlockSpec((rows_o, D), lambda i: (i, 0)),
            pl.BlockSpec((rows_o, D), lambda i: (i, 0)),
        ],
        out_specs=pl.BlockSpec((rows_o, D), lambda i: (i, 0)),
        out_shape=jax.ShapeDtypeStruct((N, D), jnp.float32),
    )(a, partials[0], partials[1])
    return out


# speedup vs baseline: 1.6418x; 1.1008x over previous
import functools
import math

import jax
import jax.numpy as jnp
from jax import lax
from jax.experimental import pallas as pl
from jax.experimental.pallas import tpu as pltpu
from jax.experimental.pallas import tpu_sc as plsc

N = 10000
E = 320000
D = 128

NC = 2
NS = 16
LANES = 16

C = 64
TCH = E // C
T0 = 2500
NZB = N // C
NZT = N - NZB * C
WBI = (NZB + NS) // NS
VPR = D // LANES


def _mm_body(h_ref, w_ref, hp_ref):
    hp_ref[...] = jnp.dot(h_ref[...], w_ref[...],
                          preferred_element_type=jnp.float32)


def _combine_body(aw_ref, p0_ref, p1_ref, o_ref):
    a = aw_ref[0, 0]
    for j in range(1, 8):
        a = a + aw_ref[0, j]
    o_ref[...] = a * (p0_ref[0] + p1_ref[0])


def _sc_body(hp_hbm, ei_hbm, out_hbm,
             hs0, hs1, hs2, hd0, hd1, hd2,
             si0, si1, si2, di0, di1, di2, dc0, dc1, dc2,
             acc,
             sg0, sg1, sg2, ss0, ss1, ss2,
             fi0, fi1, fi2, fd0, fd1, fd2):
    cid = lax.axis_index("c")
    sid = lax.axis_index("s")
    hs = (hs0, hs1, hs2)
    hd = (hd0, hd1, hd2)
    sidx = (si0, si1, si2)
    didx = (di0, di1, di2)
    dsct = (dc0, dc1, dc2)
    sg = (sg0, sg1, sg2)
    ss = (ss0, ss1, ss2)
    fi = (fi0, fi1, fi2)
    fd = (fd0, fd1, fd2)

    q0, r0 = divmod(T0, NS)
    q1, r1 = divmod(TCH - T0, NS)
    is0 = cid == 0
    my_n = jnp.where(is0,
                     q0 + (sid < r0).astype(jnp.int32),
                     q1 + (sid < r1).astype(jnp.int32))
    my_s = jnp.where(is0,
                     sid * q0 + jnp.minimum(sid, r0),
                     T0 + sid * q1 + jnp.minimum(sid, r1))

    def _idx_base(cn):
        return pl.multiple_of((my_s + cn) * C, C)

    def _zrow(e, _):
        for j in range(VPR):
            hs0[e, pl.ds(j * LANES, LANES)] = jnp.zeros((LANES,), jnp.float32)
        return 0
    lax.fori_loop(0, C, _zrow, 0)
    for r in range(WBI):
        b = r * NS + sid

        @pl.when(b < NZB)
        def _():
            pltpu.sync_copy(hs0, acc.at[pl.ds(pl.multiple_of(b * C, C), C)])

    @pl.when(sid == 0)
    def _():
        pltpu.sync_copy(hs0.at[pl.ds(0, NZT)], acc.at[pl.ds(NZB * C, NZT)])
    plsc.subcore_barrier()

    def _fetch_gidx(cn, b):
        pltpu.async_copy(ei_hbm.at[0, pl.ds(_idx_base(cn), C)], sidx[b], fi[b])
        pltpu.async_copy(ei_hbm.at[1, pl.ds(_idx_base(cn), C)], didx[b], fi[b])

    def _wait_gidx(cn, b):
        pltpu.make_async_copy(ei_hbm.at[0, pl.ds(_idx_base(cn), C)], sidx[b],
                              fi[b]).wait()
        pltpu.make_async_copy(ei_hbm.at[1, pl.ds(_idx_base(cn), C)], didx[b],
                              fi[b]).wait()

    def _issue_gather(b):
        pltpu.async_copy(hp_hbm.at[sidx[b]], hs[b], sg[b])
        pltpu.async_copy(hp_hbm.at[didx[b]], hd[b], sg[b])

    def _wait_gather(b):
        pltpu.make_async_copy(hp_hbm.at[sidx[b]], hs[b], sg[b]).wait()
        pltpu.make_async_copy(hp_hbm.at[didx[b]], hd[b], sg[b]).wait()

    def _fetch_sidx(cn, b):
        pltpu.async_copy(ei_hbm.at[1, pl.ds(_idx_base(cn), C)], dsct[b], fd[b])

    def _wait_sidx(cn, b):
        pltpu.make_async_copy(ei_hbm.at[1, pl.ds(_idx_base(cn), C)], dsct[b],
                              fd[b]).wait()

    def _compute_chunk(b):
        hsb, hdb = hs[b], hd[b]

        @plsc.parallel_loop(0, C, 1, unroll=4)
        def _(e):
            vs = [hsb[e, pl.ds(j * LANES, LANES)] for j in range(VPR)]
            vd = [hdb[e, pl.ds(j * LANES, LANES)] for j in range(VPR)]
            acc_v = vs[0] * vd[0]
            for j in range(1, VPR):
                acc_v = acc_v + vs[j] * vd[j]
            lanes = lax.iota(jnp.int32, LANES)
            for shift in (8, 4, 2, 1):
                perm = jnp.bitwise_xor(lanes, shift)
                acc_v = acc_v + acc_v.at[perm].get(mode="promise_in_bounds")
            for j in range(VPR):
                hsb[e, pl.ds(j * LANES, LANES)] = vs[j] * acc_v

    _fetch_gidx(0, 0)
    _fetch_gidx(1, 1)
    _fetch_gidx(2, 2)
    _fetch_sidx(0, 0)
    _fetch_sidx(1, 1)
    _wait_gidx(0, 0)
    _wait_gidx(1, 1)
    _issue_gather(0)
    _issue_gather(1)

    def _outer(i, _):
        for p in range(3):
            b = p
            b2 = (p + 2) % 3
            g = 3 * i + p

            @pl.when(g < my_n)
            def _():
                _wait_gather(b)

                @pl.when(g + 3 < my_n)
                def _():
                    _fetch_gidx(g + 3, b)
                _compute_chunk(b)
                _wait_sidx(g, b)
                pltpu.async_copy(hs[b], acc.at[dsct[b]], ss[b], add=True)

            @pl.when(g + 2 < my_n)
            def _():
                if p == 0:
                    @pl.when(i > 0)
                    def _():
                        pltpu.make_async_copy(hs[b2], acc.at[dsct[b2]],
                                              ss[b2]).wait()
                else:
                    pltpu.make_async_copy(hs[b2], acc.at[dsct[b2]],
                                          ss[b2]).wait()
                _wait_gidx(g + 2, b2)
                _issue_gather(b2)
                _fetch_sidx(g + 2, b2)
        return 0

    lax.fori_loop(0, (my_n + 2) // 3, _outer, 0)
    for b in range(3):
        pltpu.make_async_copy(hs[b], acc.at[dsct[b]], ss[b]).wait()

    plsc.subcore_barrier()
    for r in range(WBI):
        b = r * NS + sid

        @pl.when(b < NZB)
        def _():
            base = pl.multiple_of(b * C, C)
            pltpu.sync_copy(acc.at[pl.ds(base, C)],
                            out_hbm.at[cid, pl.ds(base, C)])

    @pl.when(sid == 0)
    def _():
        pltpu.sync_copy(acc.at[pl.ds(NZB * C, NZT)],
                        out_hbm.at[cid, pl.ds(NZB * C, NZT)])


def kernel(h, edge_index, W, attention_w):
    mm_grid = 10
    rows = N // mm_grid
    hp = pl.pallas_call(
        _mm_body,
        grid=(mm_grid,),
        in_specs=[
            pl.BlockSpec((rows, D), lambda i: (i, 0)),
            pl.BlockSpec((D, D), lambda i: (0, 0)),
        ],
        out_specs=pl.BlockSpec((rows, D), lambda i: (i, 0)),
        out_shape=jax.ShapeDtypeStruct((N, D), jnp.float32),
    )(h, W)

    mesh = plsc.VectorSubcoreMesh(core_axis_name="c", subcore_axis_name="s",
                                  num_cores=NC, num_subcores=NS)
    partials = pl.kernel(
        _sc_body,
        out_type=jax.ShapeDtypeStruct((NC, N, D), jnp.float32),
        mesh=mesh,
        scratch_types=(
            [pltpu.VMEM((C, D), jnp.float32)] * 6
            + [pltpu.VMEM((C,), jnp.int32)] * 9
            + [pltpu.VMEM_SHARED((N, D), jnp.float32)]
            + [pltpu.SemaphoreType.DMA] * 12
        ),
    )(hp, edge_index)

    grid = 10
    rows_o = N // grid
    out = pl.pallas_call(
        _combine_body,
        grid=(grid,),
        in_specs=[
            pl.BlockSpec((1, 8), lambda i: (0, 0), memory_space=pltpu.SMEM),
            pl.BlockSpec((1, rows_o, D), lambda i: (0, i, 0)),
            pl.BlockSpec((1, rows_o, D), lambda i: (1, i, 0)),
        ],
        out_specs=pl.BlockSpec((rows_o, D), lambda i: (i, 0)),
        out_shape=jax.ShapeDtypeStruct((N, D), jnp.float32),
    )(attention_w, partials, partials)
    return out
